# baseline (speedup 1.0000x reference)
import functools

import jax
import jax.numpy as jnp
from jax import lax
from jax.experimental import pallas as pl
from jax.experimental.pallas import tpu as pltpu
from jax.experimental.pallas import tpu_sc as plsc

_NUM_CORES = 2
_NUM_SUBCORES = 16
_NUM_WORKERS = _NUM_CORES * _NUM_SUBCORES
_CHUNK = 128
_NBUF = 4
_KBLK = 20
_BR = 1024
_DW = 16


def _round_up(a, b):
    return (a + b - 1) // b * b


@functools.cache
def _make_deg_kernel(e_pad, n_pad):
    epw = e_pad // _NUM_WORKERS
    n_chunks = epw // _CHUNK
    stripe = n_pad // _NUM_SUBCORES
    groups = n_chunks // 8
    mesh = plsc.VectorSubcoreMesh(core_axis_name="c", subcore_axis_name="s")

    @functools.partial(
        pl.kernel,
        out_type=jax.ShapeDtypeStruct((_NUM_CORES, n_pad, _DW), jnp.float32),
        mesh=mesh,
        scratch_types=[
            pltpu.VMEM((n_chunks, _CHUNK), jnp.int32),
            pltpu.VMEM((_CHUNK, _DW), jnp.float32),
            pltpu.VMEM_SHARED((n_pad, _DW), jnp.float32),
            pltpu.SemaphoreType.DMA,
        ],
        compiler_params=pltpu.CompilerParams(use_tc_tiling_on_sc=False),
    )
    def deg_kernel(dst_hbm, ones_hbm, zeros_hbm, out_hbm,
                   didx_v, ones_v, acc_sh, sem):
        cid = lax.axis_index("c")
        sid = lax.axis_index("s")
        wid = sid * _NUM_CORES + cid
        lo = sid * stripe
        pltpu.sync_copy(zeros_hbm, acc_sh.at[pl.ds(lo, stripe)])
        pltpu.sync_copy(ones_hbm, ones_v)
        pltpu.sync_copy(dst_hbm.at[pl.ds(wid * n_chunks, n_chunks)], didx_v)
        plsc.subcore_barrier()

        def body(gr, carry):
            for b in range(8):
                g = gr * 8 + b
                pltpu.async_copy(ones_v, acc_sh.at[didx_v.at[g]], sem,
                                 add=True)
            for b in range(8):
                g = gr * 8 + b
                pltpu.make_async_copy(ones_v, acc_sh.at[didx_v.at[g]],
                                      sem).wait()
            return carry

        lax.fori_loop(0, groups, body, 0)
        plsc.subcore_barrier()
        pltpu.sync_copy(acc_sh.at[pl.ds(lo, stripe)],
                        out_hbm.at[cid, pl.ds(lo, stripe)])

    return deg_kernel


@functools.cache
def _make_scatter_kernel(e_pad, n_pad, hf):
    eps = e_pad // _NUM_SUBCORES
    n_chunks = eps // _CHUNK
    n_blocks = n_chunks // _KBLK
    n_groups = _KBLK // _NBUF
    stripe = n_pad // _NUM_SUBCORES
    mesh = plsc.VectorSubcoreMesh(core_axis_name="c", subcore_axis_name="s")

    @functools.partial(
        pl.kernel,
        out_type=jax.ShapeDtypeStruct((_NUM_CORES, n_pad, hf), jnp.float32),
        mesh=mesh,
        scratch_types=[
            pltpu.VMEM((2, _KBLK, _CHUNK), jnp.int32),
            pltpu.VMEM((2, _KBLK, _CHUNK), jnp.int32),
            pltpu.VMEM((_NBUF, _CHUNK, hf), jnp.float32),
            pltpu.VMEM_SHARED((n_pad, hf), jnp.float32),
            pltpu.VMEM_SHARED((n_pad, hf), jnp.float32),
        ]
        + [pltpu.SemaphoreType.DMA] * (2 * _NBUF + 2),
        compiler_params=pltpu.CompilerParams(use_tc_tiling_on_sc=False),
    )
    def scatter_kernel(g_hbm, src_hbm, dst_hbm, zeros_hbm, out_hbm,
                       sidx_v, didx_v, rows_v, table_sh, acc_sh, *sems):
        gsem = sems[:_NBUF]
        ssem = sems[_NBUF:2 * _NBUF]
        isem = sems[2 * _NBUF:]
        cid = lax.axis_index("c")
        sid = lax.axis_index("s")
        lo = sid * stripe
        base = sid * n_chunks
        pltpu.sync_copy(zeros_hbm, acc_sh.at[pl.ds(lo, stripe)])
        pltpu.sync_copy(g_hbm.at[pl.ds(cid * n_pad + lo, stripe)],
                        table_sh.at[pl.ds(lo, stripe)])
        for p in range(2):
            pltpu.async_copy(src_hbm.at[pl.ds(base + p * _KBLK, _KBLK)],
                             sidx_v.at[p], isem[p])
            pltpu.async_copy(dst_hbm.at[pl.ds(base + p * _KBLK, _KBLK)],
                             didx_v.at[p], isem[p])
        plsc.subcore_barrier()

        def do_block(kb, p):
            blk = base + kb * _KBLK
            pltpu.make_async_copy(src_hbm.at[pl.ds(blk, _KBLK)],
                                  sidx_v.at[p], isem[p]).wait()
            pltpu.make_async_copy(dst_hbm.at[pl.ds(blk, _KBLK)],
                                  didx_v.at[p], isem[p]).wait()
            for b in range(_NBUF):
                pltpu.async_copy(table_sh.at[sidx_v.at[p, b]], rows_v.at[b],
                                 gsem[b])

            def group(gr, carry):
                for b in range(_NBUF):
                    c = gr * _NBUF + b
                    pltpu.make_async_copy(table_sh.at[sidx_v.at[p, c]],
                                          rows_v.at[b], gsem[b]).wait()
                    pltpu.async_copy(rows_v.at[b], acc_sh.at[didx_v.at[p, c]],
                                     ssem[b], add=True)
                for b in range(_NBUF):
                    c = (gr + 1) * _NBUF + b

                    @pl.when(c < _KBLK)
                    def _():
                        pltpu.make_async_copy(
                            rows_v.at[b], acc_sh.at[didx_v.at[p, c - _NBUF]],
                            ssem[b]).wait()
                        pltpu.async_copy(table_sh.at[sidx_v.at[p, c]],
                                         rows_v.at[b], gsem[b])
                return carry

            lax.fori_loop(0, n_groups, group, 0)
            last = (n_groups - 1) * _NBUF
            for b in range(_NBUF):
                pltpu.make_async_copy(rows_v.at[b],
                                      acc_sh.at[didx_v.at[p, last + b]],
                                      ssem[b]).wait()

            @pl.when(kb + 2 < n_blocks)
            def _():
                nxt = base + (kb + 2) * _KBLK
                pltpu.async_copy(src_hbm.at[pl.ds(nxt, _KBLK)],
                                 sidx_v.at[p], isem[p])
                pltpu.async_copy(dst_hbm.at[pl.ds(nxt, _KBLK)],
                                 didx_v.at[p], isem[p])

        def pair(i, carry):
            do_block(2 * i, 0)
            do_block(2 * i + 1, 1)
            return carry

        lax.fori_loop(0, n_blocks // 2, pair, 0)
        plsc.subcore_barrier()
        pltpu.sync_copy(acc_sh.at[pl.ds(lo, stripe)],
                        out_hbm.at[cid, pl.ds(lo, stripe)])

    return scatter_kernel


def _tc_layer1(x_pad, W1, degparts):
    n_pad, nf = x_pad.shape
    nh = W1.shape[1]
    hf = nh // 2

    def body(x_ref, w_ref, dp_ref, g1_ref):
        deg = dp_ref[0, :, 0] + dp_ref[1, :, 0] + 1.0
        dis = lax.rsqrt(deg)
        z1 = jnp.dot(x_ref[...], w_ref[...], preferred_element_type=jnp.float32)
        g1 = z1 * dis[:, None]
        g1_ref[0] = g1[:, :hf]
        g1_ref[1] = g1[:, hf:]

    return pl.pallas_call(
        body,
        grid=(n_pad // _BR,),
        in_specs=[
            pl.BlockSpec((_BR, nf), lambda i: (i, 0)),
            pl.BlockSpec((nf, nh), lambda i: (0, 0)),
            pl.BlockSpec((_NUM_CORES, _BR, _DW), lambda i: (0, i, 0)),
        ],
        out_specs=pl.BlockSpec((_NUM_CORES, _BR, hf), lambda i: (0, i, 0)),
        out_shape=jax.ShapeDtypeStruct((_NUM_CORES, n_pad, hf), jnp.float32),
    )(x_pad, W1, degparts)


def _tc_layer2(s1, g1, degparts, b1, W2, n_real):
    _, n_pad, hf1 = s1.shape
    nh = 2 * hf1
    nc = W2.shape[1]

    def body(sp_ref, g1_ref, dp_ref, b1_ref, w2_ref, g2_ref):
        i = pl.program_id(0)
        deg = dp_ref[0, :, 0] + dp_ref[1, :, 0] + 1.0
        dis = lax.rsqrt(deg)
        s = jnp.concatenate(
            [sp_ref[0] + g1_ref[0], sp_ref[1] + g1_ref[1]], axis=1)
        h = jnp.maximum(s * dis[:, None] + b1_ref[...], 0.0)
        z2 = jnp.dot(h, w2_ref[...], preferred_element_type=jnp.float32)
        g2 = z2 * dis[:, None]
        rows = i * _BR + lax.broadcasted_iota(jnp.int32, (_BR, nc), 0)
        g2 = jnp.where(rows < n_real, g2, 0.0)
        g2_ref[0] = g2[:, :nc // 2]
        g2_ref[1] = g2[:, nc // 2:]

    return pl.pallas_call(
        body,
        grid=(n_pad // _BR,),
        in_specs=[
            pl.BlockSpec((_NUM_CORES, _BR, hf1), lambda i: (0, i, 0)),
            pl.BlockSpec((_NUM_CORES, _BR, hf1), lambda i: (0, i, 0)),
            pl.BlockSpec((_NUM_CORES, _BR, _DW), lambda i: (0, i, 0)),
            pl.BlockSpec((1, nh), lambda i: (0, 0)),
            pl.BlockSpec((nh, nc), lambda i: (0, 0)),
        ],
        out_specs=pl.BlockSpec((_NUM_CORES, _BR, nc // 2), lambda i: (0, i, 0)),
        out_shape=jax.ShapeDtypeStruct((_NUM_CORES, n_pad, nc // 2), jnp.float32),
    )(s1, g1, degparts, b1, W2)


def _tc_out(s2, g2, degparts, b2, n_real):
    _, n_pad, hf2 = s2.shape
    nc = 2 * hf2

    def body(sp_ref, g2_ref, dp_ref, b2_ref, o_ref):
        deg = dp_ref[0, :, 0] + dp_ref[1, :, 0] + 1.0
        dis = lax.rsqrt(deg)
        s = jnp.concatenate(
            [sp_ref[0] + g2_ref[0], sp_ref[1] + g2_ref[1]], axis=1)
        o = s * dis[:, None] + b2_ref[...]
        m = jnp.max(o, axis=1, keepdims=True)
        xs = o - m
        lse = jnp.log(jnp.sum(jnp.exp(xs), axis=1, keepdims=True))
        o_ref[...] = xs - lse

    return pl.pallas_call(
        body,
        grid=(n_pad // _BR,),
        in_specs=[
            pl.BlockSpec((_NUM_CORES, _BR, hf2), lambda i: (0, i, 0)),
            pl.BlockSpec((_NUM_CORES, _BR, hf2), lambda i: (0, i, 0)),
            pl.BlockSpec((_NUM_CORES, _BR, _DW), lambda i: (0, i, 0)),
            pl.BlockSpec((1, nc), lambda i: (0, 0)),
        ],
        out_specs=pl.BlockSpec((_BR, nc), lambda i: (i, 0)),
        out_shape=jax.ShapeDtypeStruct((n_real, nc), jnp.float32),
    )(s2, g2, degparts, b2)


@jax.jit
def kernel(x, edge_index, W1, b1, W2, b2):
    n, nf = x.shape
    e = edge_index.shape[1]
    nh = W1.shape[1]
    nc = W2.shape[1]

    e_pad = _round_up(e, _NUM_SUBCORES * _CHUNK * 2 * _KBLK * 8 // 8)
    e_pad = _round_up(e_pad, _NUM_WORKERS * _CHUNK * 8)
    n_pad = _round_up(n + 1, _BR)

    src = edge_index[0]
    dst = edge_index[1]
    pad = e_pad - e
    spread = n + jax.lax.rem(jnp.arange(pad, dtype=jnp.int32),
                             jnp.int32(n_pad - n))
    srcp = jnp.concatenate([src, spread]).reshape(-1, _CHUNK)
    dstp = jnp.concatenate([dst, spread]).reshape(-1, _CHUNK)
    x_pad = jnp.pad(x, ((0, n_pad - n), (0, 0)))

    ones_dw = jnp.ones((_CHUNK, _DW), jnp.float32)
    zeros_dw = jnp.zeros((n_pad // _NUM_SUBCORES, _DW), jnp.float32)
    degparts = _make_deg_kernel(e_pad, n_pad)(dstp, ones_dw, zeros_dw)

    g1 = _tc_layer1(x_pad, W1, degparts)
    zeros1 = jnp.zeros((n_pad // _NUM_SUBCORES, nh // 2), jnp.float32)
    s1 = _make_scatter_kernel(e_pad, n_pad, nh // 2)(
        g1.reshape(2 * n_pad, nh // 2), srcp, dstp, zeros1)

    g2 = _tc_layer2(s1, g1, degparts, b1.reshape(1, nh), W2, n)
    zeros2 = jnp.zeros((n_pad // _NUM_SUBCORES, nc // 2), jnp.float32)
    s2 = _make_scatter_kernel(e_pad, n_pad, nc // 2)(
        g2.reshape(2 * n_pad, nc // 2), srcp, dstp, zeros2)

    return _tc_out(s2, g2, degparts, b2.reshape(1, nc), n)


# speedup vs baseline: 1.3753x; 1.3753x over previous
import functools

import jax
import jax.numpy as jnp
from jax import lax
from jax.experimental import pallas as pl
from jax.experimental.pallas import tpu as pltpu
from jax.experimental.pallas import tpu_sc as plsc

_NUM_CORES = 2
_NUM_SUBCORES = 16
_NUM_WORKERS = _NUM_CORES * _NUM_SUBCORES
_CHUNK = 128
_NBUF = 4
_BR = 1024
_DW = 16


def _round_up(a, b):
    return (a + b - 1) // b * b


@functools.cache
def _make_deg_kernel(e_pad, n_pad):
    epw = e_pad // _NUM_WORKERS
    n_chunks = epw // _CHUNK
    stripe = n_pad // _NUM_SUBCORES
    groups = n_chunks // 8
    mesh = plsc.VectorSubcoreMesh(core_axis_name="c", subcore_axis_name="s")

    @functools.partial(
        pl.kernel,
        out_type=jax.ShapeDtypeStruct((_NUM_CORES, n_pad, _DW), jnp.float32),
        mesh=mesh,
        scratch_types=[
            pltpu.VMEM((n_chunks, _CHUNK), jnp.int32),
            pltpu.VMEM((_CHUNK, _DW), jnp.float32),
            pltpu.VMEM_SHARED((n_pad, _DW), jnp.float32),
            pltpu.SemaphoreType.DMA,
        ],
        compiler_params=pltpu.CompilerParams(use_tc_tiling_on_sc=False),
    )
    def deg_kernel(dst_hbm, ones_hbm, zeros_hbm, out_hbm,
                   didx_v, ones_v, acc_sh, sem):
        cid = lax.axis_index("c")
        sid = lax.axis_index("s")
        wid = sid * _NUM_CORES + cid
        lo = sid * stripe
        pltpu.sync_copy(zeros_hbm, acc_sh.at[pl.ds(lo, stripe)])
        pltpu.sync_copy(ones_hbm, ones_v)
        pltpu.sync_copy(dst_hbm.at[pl.ds(wid * n_chunks, n_chunks)], didx_v)
        plsc.subcore_barrier()

        def body(gr, carry):
            for b in range(8):
                g = gr * 8 + b
                pltpu.async_copy(ones_v, acc_sh.at[didx_v.at[g]], sem,
                                 add=True)
            for b in range(8):
                g = gr * 8 + b
                pltpu.make_async_copy(ones_v, acc_sh.at[didx_v.at[g]],
                                      sem).wait()
            return carry

        lax.fori_loop(0, groups, body, 0)
        plsc.subcore_barrier()
        pltpu.sync_copy(acc_sh.at[pl.ds(lo, stripe)],
                        out_hbm.at[cid, pl.ds(lo, stripe)])

    return deg_kernel


@functools.cache
def _make_scatter_kernel(e_pad, n_pad, hf, nbuf=_NBUF):
    eps = e_pad // _NUM_SUBCORES
    n_chunks = eps // _CHUNK
    n_groups = n_chunks // nbuf
    stripe = n_pad // _NUM_SUBCORES
    mesh = plsc.VectorSubcoreMesh(core_axis_name="c", subcore_axis_name="s")

    @functools.partial(
        pl.kernel,
        out_type=jax.ShapeDtypeStruct((_NUM_CORES, n_pad, hf), jnp.float32),
        mesh=mesh,
        scratch_types=[
            pltpu.VMEM((n_chunks, _CHUNK), jnp.int32),
            pltpu.VMEM((n_chunks, _CHUNK), jnp.int32),
            pltpu.VMEM((nbuf, _CHUNK, hf), jnp.float32),
            pltpu.VMEM_SHARED((n_pad, hf), jnp.float32),
        ]
        + [pltpu.SemaphoreType.DMA] * (2 * nbuf),
        compiler_params=pltpu.CompilerParams(use_tc_tiling_on_sc=False),
    )
    def scatter_kernel(g_hbm, src2_hbm, dst_hbm, zeros_hbm, out_hbm,
                       sidx_v, didx_v, rows_v, acc_sh, *sems):
        gsem = sems[:nbuf]
        ssem = sems[nbuf:]
        cid = lax.axis_index("c")
        sid = lax.axis_index("s")
        lo = sid * stripe
        pltpu.sync_copy(zeros_hbm, acc_sh.at[pl.ds(lo, stripe)])
        pltpu.sync_copy(src2_hbm.at[cid, pl.ds(sid * n_chunks, n_chunks)],
                        sidx_v)
        pltpu.sync_copy(dst_hbm.at[pl.ds(sid * n_chunks, n_chunks)], didx_v)
        plsc.subcore_barrier()

        for b in range(nbuf):
            pltpu.async_copy(g_hbm.at[sidx_v.at[b]], rows_v.at[b], gsem[b])

        def body(gr, carry):
            for b in range(nbuf):
                g = gr * nbuf + b
                pltpu.make_async_copy(g_hbm.at[sidx_v.at[g]], rows_v.at[b],
                                      gsem[b]).wait()
                pltpu.async_copy(rows_v.at[b], acc_sh.at[didx_v.at[g]],
                                 ssem[b], add=True)
            for b in range(nbuf):
                g = (gr + 1) * nbuf + b

                @pl.when(g < n_chunks)
                def _():
                    pltpu.make_async_copy(rows_v.at[b],
                                          acc_sh.at[didx_v.at[g - nbuf]],
                                          ssem[b]).wait()
                    pltpu.async_copy(g_hbm.at[sidx_v.at[g]], rows_v.at[b],
                                     gsem[b])
            return carry

        lax.fori_loop(0, n_groups, body, 0)
        last = (n_groups - 1) * nbuf
        for b in range(nbuf):
            pltpu.make_async_copy(rows_v.at[b],
                                  acc_sh.at[didx_v.at[last + b]],
                                  ssem[b]).wait()
        plsc.subcore_barrier()
        pltpu.sync_copy(acc_sh.at[pl.ds(lo, stripe)],
                        out_hbm.at[cid, pl.ds(lo, stripe)])

    return scatter_kernel


def _tc_layer1(x_pad, W1, degparts):
    n_pad, nf = x_pad.shape
    nh = W1.shape[1]

    def body(x_ref, w_ref, dp_ref, g1_ref):
        deg = dp_ref[0, :, 0] + dp_ref[1, :, 0] + 1.0
        dis = lax.rsqrt(deg)
        z1 = jnp.dot(x_ref[...], w_ref[...], preferred_element_type=jnp.float32)
        g1_ref[...] = z1 * dis[:, None]

    return pl.pallas_call(
        body,
        grid=(n_pad // _BR,),
        in_specs=[
            pl.BlockSpec((_BR, nf), lambda i: (i, 0)),
            pl.BlockSpec((nf, nh), lambda i: (0, 0)),
            pl.BlockSpec((_NUM_CORES, _BR, _DW), lambda i: (0, i, 0)),
        ],
        out_specs=pl.BlockSpec((_BR, nh), lambda i: (i, 0)),
        out_shape=jax.ShapeDtypeStruct((n_pad, nh), jnp.float32),
    )(x_pad, W1, degparts)


def _tc_layer2(s1, g1, degparts, b1, W2, n_real):
    _, n_pad, hf1 = s1.shape
    nh = 2 * hf1
    nc = W2.shape[1]

    def body(sp_ref, g1_ref, dp_ref, b1_ref, w2_ref, g2_ref):
        i = pl.program_id(0)
        deg = dp_ref[0, :, 0] + dp_ref[1, :, 0] + 1.0
        dis = lax.rsqrt(deg)
        s = jnp.concatenate([sp_ref[0], sp_ref[1]], axis=1) + g1_ref[...]
        h = jnp.maximum(s * dis[:, None] + b1_ref[...], 0.0)
        z2 = jnp.dot(h, w2_ref[...], preferred_element_type=jnp.float32)
        g2 = z2 * dis[:, None]
        rows = i * _BR + lax.broadcasted_iota(jnp.int32, (_BR, nc), 0)
        g2 = jnp.where(rows < n_real, g2, 0.0)
        g2_ref[0] = g2[:, :nc // 2]
        g2_ref[1] = g2[:, nc // 2:]

    return pl.pallas_call(
        body,
        grid=(n_pad // _BR,),
        in_specs=[
            pl.BlockSpec((_NUM_CORES, _BR, hf1), lambda i: (0, i, 0)),
            pl.BlockSpec((_BR, nh), lambda i: (i, 0)),
            pl.BlockSpec((_NUM_CORES, _BR, _DW), lambda i: (0, i, 0)),
            pl.BlockSpec((1, nh), lambda i: (0, 0)),
            pl.BlockSpec((nh, nc), lambda i: (0, 0)),
        ],
        out_specs=pl.BlockSpec((_NUM_CORES, _BR, nc // 2), lambda i: (0, i, 0)),
        out_shape=jax.ShapeDtypeStruct((_NUM_CORES, n_pad, nc // 2), jnp.float32),
    )(s1, g1, degparts, b1, W2)


def _tc_out(s2, g2, degparts, b2, n_real):
    _, n_pad, hf2 = s2.shape
    nc = 2 * hf2

    def body(sp_ref, g2_ref, dp_ref, b2_ref, o_ref):
        deg = dp_ref[0, :, 0] + dp_ref[1, :, 0] + 1.0
        dis = lax.rsqrt(deg)
        s = jnp.concatenate(
            [sp_ref[0] + g2_ref[0], sp_ref[1] + g2_ref[1]], axis=1)
        o = s * dis[:, None] + b2_ref[...]
        m = jnp.max(o, axis=1, keepdims=True)
        xs = o - m
        lse = jnp.log(jnp.sum(jnp.exp(xs), axis=1, keepdims=True))
        o_ref[...] = xs - lse

    return pl.pallas_call(
        body,
        grid=(n_pad // _BR,),
        in_specs=[
            pl.BlockSpec((_NUM_CORES, _BR, hf2), lambda i: (0, i, 0)),
            pl.BlockSpec((_NUM_CORES, _BR, hf2), lambda i: (0, i, 0)),
            pl.BlockSpec((_NUM_CORES, _BR, _DW), lambda i: (0, i, 0)),
            pl.BlockSpec((1, nc), lambda i: (0, 0)),
        ],
        out_specs=pl.BlockSpec((_BR, nc), lambda i: (i, 0)),
        out_shape=jax.ShapeDtypeStruct((n_real, nc), jnp.float32),
    )(s2, g2, degparts, b2)


@jax.jit
def kernel(x, edge_index, W1, b1, W2, b2):
    n, nf = x.shape
    e = edge_index.shape[1]
    nh = W1.shape[1]
    nc = W2.shape[1]

    e_pad = _round_up(e, _NUM_WORKERS * _CHUNK * 8)
    n_pad = _round_up(n + 1, _BR)

    src = edge_index[0]
    dst = edge_index[1]
    pad = e_pad - e
    spread = n + jax.lax.rem(jnp.arange(pad, dtype=jnp.int32),
                             jnp.int32(n_pad - n))
    srcp = jnp.concatenate([src, spread]).reshape(-1, _CHUNK)
    dstp = jnp.concatenate([dst, spread]).reshape(-1, _CHUNK)
    src2a = jnp.stack([2 * srcp, 2 * srcp + 1])
    src2b = jnp.stack([srcp, srcp + n_pad])
    x_pad = jnp.pad(x, ((0, n_pad - n), (0, 0)))

    ones_dw = jnp.ones((_CHUNK, _DW), jnp.float32)
    zeros_dw = jnp.zeros((n_pad // _NUM_SUBCORES, _DW), jnp.float32)
    degparts = _make_deg_kernel(e_pad, n_pad)(dstp, ones_dw, zeros_dw)

    g1 = _tc_layer1(x_pad, W1, degparts)
    zeros1 = jnp.zeros((n_pad // _NUM_SUBCORES, nh // 2), jnp.float32)
    s1 = _make_scatter_kernel(e_pad, n_pad, nh // 2)(
        g1.reshape(2 * n_pad, nh // 2), src2a, dstp, zeros1)

    g2 = _tc_layer2(s1, g1, degparts, b1.reshape(1, nh), W2, n)
    zeros2 = jnp.zeros((n_pad // _NUM_SUBCORES, nc // 2), jnp.float32)
    s2 = _make_scatter_kernel(e_pad, n_pad, nc // 2, nbuf=8)(
        g2.reshape(2 * n_pad, nc // 2), src2b, dstp, zeros2)

    return _tc_out(s2, g2, degparts, b2.reshape(1, nc), n)
